# Initial kernel scaffold; baseline (speedup 1.0000x reference)
import jax
import jax.numpy as jnp
from jax.experimental import pallas as pl


def kernel(x, in_adj, edge_index, W_heads, a_heads, b_heads, W_out, a_out, b_out):
    raise NotImplementedError("write your pallas kernel here")


# speedup vs baseline: 14.2496x; 14.2496x over previous
import functools

import jax
import jax.numpy as jnp
from jax.experimental import pallas as pl
from jax.experimental.pallas import tpu as pltpu

N = 2048
D = 128
NHEAD = 8
NHID = 128
NCLASS = 10
ALPHA = 0.2

RB = 256
CB = 256
NRB = N // RB
NCB = N // CB


def _leaky(v):
    return jnp.where(v >= 0, v, ALPHA * v)


def _proj_kernel(x_ref, w_ref, a1_ref, a2_ref, h_ref, as_ref, ad_ref, hsum_ref):
    r = pl.program_id(0)
    xb = x_ref[...]
    as_cols = []
    ad_cols = []
    hs = []
    for i in range(NHEAD):
        h = jnp.dot(xb, w_ref[i], preferred_element_type=jnp.float32)
        h_ref[i] = h
        as_cols.append(jnp.dot(h, a1_ref[i][:, None],
                               preferred_element_type=jnp.float32))
        ad_cols.append(jnp.dot(h, a2_ref[i][:, None],
                               preferred_element_type=jnp.float32))
        hs.append(jnp.sum(h, axis=0, keepdims=True))
    as_ref[...] = jnp.concatenate(as_cols, axis=1)
    ad_ref[...] = jnp.concatenate(ad_cols, axis=1)
    part = jnp.concatenate(hs, axis=0)

    @pl.when(r == 0)
    def _init():
        hsum_ref[...] = part

    @pl.when(r != 0)
    def _acc():
        hsum_ref[...] += part


def _heads_kernel(in_adj_ref, h_ref, as_ref, ad_ref, hsum_ref, b_ref,
                  adj_ref, h1_ref, acc_ref, den_ref):
    r = pl.program_id(0)
    c = pl.program_id(1)
    rows = r * RB + jax.lax.broadcasted_iota(jnp.int32, (RB, CB), 0)
    cols = c * CB + jax.lax.broadcasted_iota(jnp.int32, (RB, CB), 1)
    adj = in_adj_ref[...] + jnp.where(rows == cols, 1.0, 0.0)
    adj_ref[...] = adj

    asb = as_ref[...]
    adb = ad_ref[...]
    dens = []
    for i in range(NHEAD):
        e = _leaky(asb[:, i][:, None] + adb[:, i][None, :])
        w = jnp.exp(adj * e) - 1.0
        dens.append(jnp.sum(w, axis=1, keepdims=True))
        contrib = jnp.dot(w, h_ref[i], preferred_element_type=jnp.float32)

        @pl.when(c == 0)
        def _init(i=i, contrib=contrib):
            acc_ref[i] = contrib

        @pl.when(c != 0)
        def _acc(i=i, contrib=contrib):
            acc_ref[i] += contrib

    den_part = jnp.concatenate(dens, axis=1)

    @pl.when(c == 0)
    def _dinit():
        den_ref[...] = den_part

    @pl.when(c != 0)
    def _dacc():
        den_ref[...] += den_part

    @pl.when(c == NCB - 1)
    def _finalize():
        outs = []
        for i in range(NHEAD):
            numer = hsum_ref[i][None, :] + acc_ref[i]
            den = float(N) + den_ref[:, i][:, None]
            o = numer / den + b_ref[i][None, :]
            outs.append(jnp.where(o > 0, o, jnp.exp(o) - 1.0))
        h1_ref[...] = jnp.concatenate(outs, axis=1)


def _out_proj_kernel(h1_ref, wout_ref, h2_ref, hsum2_ref):
    h2 = jnp.dot(h1_ref[...], wout_ref[...], preferred_element_type=jnp.float32)
    h2_ref[...] = h2
    hsum2_ref[0] = jnp.sum(h2, axis=0, keepdims=True)


def _final_kernel(in_adj_ref, h2r_ref, h2c_ref, hsum2_ref, a1_ref, a2_ref,
                  b_ref, out_ref, acc_ref, den_ref):
    r = pl.program_id(0)
    c = pl.program_id(1)
    rows = r * RB + jax.lax.broadcasted_iota(jnp.int32, (RB, CB), 0)
    cols = c * CB + jax.lax.broadcasted_iota(jnp.int32, (RB, CB), 1)
    adj = in_adj_ref[...] + jnp.where(rows == cols, 1.0, 0.0)

    h2r = h2r_ref[...]
    h2c = h2c_ref[...]
    asr = jnp.dot(h2r, a1_ref[...], preferred_element_type=jnp.float32)
    adc = jnp.dot(h2c, a2_ref[...], preferred_element_type=jnp.float32)
    e = _leaky(asr + adc[:, 0][None, :])
    w = jnp.exp(adj * e) - 1.0
    den_part = jnp.sum(w, axis=1, keepdims=True)
    contrib = jnp.dot(w, h2c, preferred_element_type=jnp.float32)

    @pl.when(c == 0)
    def _init():
        acc_ref[...] = contrib
        den_ref[...] = jnp.broadcast_to(den_part, (RB, 128))

    @pl.when(c != 0)
    def _acc():
        acc_ref[...] += contrib
        den_ref[...] += jnp.broadcast_to(den_part, (RB, 128))

    @pl.when(c == NCB - 1)
    def _finalize():
        hsum2 = jnp.sum(hsum2_ref[...], axis=0)
        numer = hsum2 + acc_ref[...]
        den = float(N) + den_ref[:, 0][:, None]
        o = numer / den + b_ref[...]
        lane = jax.lax.broadcasted_iota(jnp.int32, (RB, 128), 1)
        valid = lane < NCLASS
        om = jnp.where(valid, o, -jnp.inf)
        mx = jnp.max(om, axis=1, keepdims=True)
        ex = jnp.where(valid, jnp.exp(om - mx), 0.0)
        lse = jnp.log(jnp.sum(ex, axis=1, keepdims=True)) + mx
        out_ref[...] = jnp.where(valid, o - lse, 0.0)


def kernel(x, in_adj, edge_index, W_heads, a_heads, b_heads, W_out, a_out, b_out):
    del edge_index
    a1 = a_heads[:, :D, 0]
    a2 = a_heads[:, D:, 0]

    h, as_, ad_, hsum = pl.pallas_call(
        _proj_kernel,
        grid=(NRB,),
        in_specs=[
            pl.BlockSpec((RB, D), lambda r: (r, 0)),
            pl.BlockSpec((NHEAD, D, D), lambda r: (0, 0, 0)),
            pl.BlockSpec((NHEAD, D), lambda r: (0, 0)),
            pl.BlockSpec((NHEAD, D), lambda r: (0, 0)),
        ],
        out_specs=[
            pl.BlockSpec((NHEAD, RB, D), lambda r: (0, r, 0)),
            pl.BlockSpec((RB, NHEAD), lambda r: (r, 0)),
            pl.BlockSpec((RB, NHEAD), lambda r: (r, 0)),
            pl.BlockSpec((NHEAD, D), lambda r: (0, 0)),
        ],
        out_shape=[
            jax.ShapeDtypeStruct((NHEAD, N, D), jnp.float32),
            jax.ShapeDtypeStruct((N, NHEAD), jnp.float32),
            jax.ShapeDtypeStruct((N, NHEAD), jnp.float32),
            jax.ShapeDtypeStruct((NHEAD, D), jnp.float32),
        ],
        compiler_params=pltpu.CompilerParams(
            dimension_semantics=("arbitrary",)),
    )(x, W_heads, a1, a2)

    adj, h1 = pl.pallas_call(
        _heads_kernel,
        grid=(NRB, NCB),
        in_specs=[
            pl.BlockSpec((RB, CB), lambda r, c: (r, c)),
            pl.BlockSpec((NHEAD, CB, D), lambda r, c: (0, c, 0)),
            pl.BlockSpec((RB, NHEAD), lambda r, c: (r, 0)),
            pl.BlockSpec((CB, NHEAD), lambda r, c: (c, 0)),
            pl.BlockSpec((NHEAD, D), lambda r, c: (0, 0)),
            pl.BlockSpec((NHEAD, D), lambda r, c: (0, 0)),
        ],
        out_specs=[
            pl.BlockSpec((RB, CB), lambda r, c: (r, c)),
            pl.BlockSpec((RB, NHEAD * D), lambda r, c: (r, 0)),
        ],
        out_shape=[
            jax.ShapeDtypeStruct((N, N), jnp.float32),
            jax.ShapeDtypeStruct((N, NHEAD * D), jnp.float32),
        ],
        scratch_shapes=[
            pltpu.VMEM((NHEAD, RB, D), jnp.float32),
            pltpu.VMEM((RB, NHEAD), jnp.float32),
        ],
        compiler_params=pltpu.CompilerParams(
            dimension_semantics=("parallel", "arbitrary")),
    )(in_adj, h, as_, ad_, hsum, b_heads)

    wout_pad = jnp.zeros((NHEAD * D, 128), jnp.float32).at[:, :NCLASS].set(W_out)
    a1o = jnp.zeros((128, 1), jnp.float32).at[:NCLASS, 0].set(a_out[:NCLASS, 0])
    a2o = jnp.zeros((128, 1), jnp.float32).at[:NCLASS, 0].set(a_out[NCLASS:, 0])
    bo = jnp.zeros((1, 128), jnp.float32).at[0, :NCLASS].set(b_out)

    h2, hsum2 = pl.pallas_call(
        _out_proj_kernel,
        grid=(NRB,),
        in_specs=[
            pl.BlockSpec((RB, NHEAD * D), lambda r: (r, 0)),
            pl.BlockSpec((NHEAD * D, 128), lambda r: (0, 0)),
        ],
        out_specs=[
            pl.BlockSpec((RB, 128), lambda r: (r, 0)),
            pl.BlockSpec((1, 1, 128), lambda r: (r, 0, 0)),
        ],
        out_shape=[
            jax.ShapeDtypeStruct((N, 128), jnp.float32),
            jax.ShapeDtypeStruct((NRB, 1, 128), jnp.float32),
        ],
        compiler_params=pltpu.CompilerParams(
            dimension_semantics=("arbitrary",)),
    )(h1, wout_pad)

    out_pad = pl.pallas_call(
        _final_kernel,
        grid=(NRB, NCB),
        in_specs=[
            pl.BlockSpec((RB, CB), lambda r, c: (r, c)),
            pl.BlockSpec((RB, 128), lambda r, c: (r, 0)),
            pl.BlockSpec((CB, 128), lambda r, c: (c, 0)),
            pl.BlockSpec((NRB, 1, 128), lambda r, c: (0, 0, 0)),
            pl.BlockSpec((128, 1), lambda r, c: (0, 0)),
            pl.BlockSpec((128, 1), lambda r, c: (0, 0)),
            pl.BlockSpec((1, 128), lambda r, c: (0, 0)),
        ],
        out_specs=pl.BlockSpec((RB, 128), lambda r, c: (r, 0)),
        out_shape=jax.ShapeDtypeStruct((N, 128), jnp.float32),
        scratch_shapes=[
            pltpu.VMEM((RB, 128), jnp.float32),
            pltpu.VMEM((RB, 128), jnp.float32),
        ],
        compiler_params=pltpu.CompilerParams(
            dimension_semantics=("parallel", "arbitrary")),
    )(in_adj, h2, h2, hsum2, a1o, a2o, bo)

    return out_pad[:, :NCLASS], adj, x
